# baseline (speedup 1.0000x reference)
import functools

import jax
import jax.numpy as jnp
from jax import lax
from jax.experimental import pallas as pl
from jax.experimental.pallas import tpu as pltpu
from jax.experimental.pallas import tpu_sc as plsc

TEMP_INV = 1.0 / 0.07
B, L, V, TD, FD = 4096, 50, 100000, 384, 4
FDP = 16
HIST = B * L
NC, NS = 2, 16
NW = NC * NS
CH = 128
HIST_PER_W = HIST // NW
POS_PER_W = B // NW
N_HCHUNK = HIST_PER_W // CH


def _mesh():
    return plsc.VectorSubcoreMesh(core_axis_name="c", subcore_axis_name="s")


def _idx_transpose(ih):
    BB = 512

    def body(i_ref, o_ref):
        o_ref[...] = jnp.transpose(i_ref[...])[:, None, :]

    return pl.pallas_call(
        body,
        grid=(B // BB,),
        in_specs=[pl.BlockSpec((BB, L), lambda i: (i, 0))],
        out_specs=pl.BlockSpec((L, 1, BB), lambda i: (0, 0, i)),
        out_shape=jax.ShapeDtypeStruct((L, 1, B), jnp.int32),
    )(ih)


def _make_title_gather(nslots, lo, with_pos):
    per_w = nslots * B // NW
    nch = per_w // CH
    out_type = [jax.ShapeDtypeStruct((nslots, B, TD), jnp.float32)]
    if with_pos:
        out_type += [jax.ShapeDtypeStruct((B, TD), jnp.float32)]

    @functools.partial(
        pl.kernel,
        mesh=_mesh(),
        out_type=tuple(out_type),
        scratch_types=[
            pltpu.VMEM((per_w,), jnp.int32),
            pltpu.VMEM((2, CH, TD), jnp.float32),
            pltpu.SemaphoreType.DMA,
            pltpu.SemaphoreType.DMA,
            pltpu.SemaphoreType.DMA,
            pltpu.SemaphoreType.DMA,
        ],
    )
    def k(ih_hbm, *rest):
        if with_pos:
            (ip_hbm, tt_hbm, oht, opt, idx_all, rows_v,
             gsem0, gsem1, wsem0, wsem1) = rest
        else:
            (tt_hbm, oht, idx_all, rows_v, gsem0, gsem1, wsem0, wsem1) = rest
        wid = lax.axis_index("s") * NC + lax.axis_index("c")
        hbase = wid * per_w
        gsem = (gsem0, gsem1)
        wsem = (wsem0, wsem1)

        def lb(c):
            base = hbase + c * CH
            l_c = base // B
            return l_c, base - l_c * B

        for c in range(nch):
            l_c, b0 = lb(c)
            pltpu.async_copy(ih_hbm.at[lo + l_c, 0, pl.ds(b0, CH)],
                             idx_all.at[pl.ds(c * CH, CH)], gsem0)
        for c in range(nch):
            pltpu.make_async_copy(ih_hbm.at[0, 0, pl.ds(0, CH)],
                                  idx_all.at[pl.ds(0, CH)], gsem0).wait()

        def fire_gather(c, b):
            pltpu.async_copy(tt_hbm.at[idx_all.at[pl.ds(c * CH, CH)]],
                             rows_v.at[b], gsem[b])

        def wait_gather(b):
            pltpu.make_async_copy(tt_hbm.at[pl.ds(0, CH)], rows_v.at[b],
                                  gsem[b]).wait()

        def fire_wb(c, b):
            l_c, b0 = lb(c)
            pltpu.async_copy(rows_v.at[b], oht.at[l_c, pl.ds(b0, CH)],
                             wsem[b])

        def wait_wb(b):
            pltpu.make_async_copy(rows_v.at[b], oht.at[0, pl.ds(0, CH)],
                                  wsem[b]).wait()

        fire_gather(0, 0)

        def body(g2, carry):
            for b in (0, 1):
                cg = 2 * g2 + b
                nb = 1 - b

                @pl.when(cg + 1 < nch)
                def _fire_next():
                    @pl.when(cg >= 1)
                    def _drain_prev_wb():
                        wait_wb(nb)

                    fire_gather(cg + 1, nb)

                wait_gather(b)
                fire_wb(cg, b)
            return carry

        lax.fori_loop(0, nch // 2, body, 0)
        if nch % 2:
            wait_gather(0)
            fire_wb(nch - 1, 0)
        wait_wb(0)
        if nch >= 2:
            wait_wb(1)

        if with_pos:
            pltpu.sync_copy(ip_hbm.at[pl.ds(wid * POS_PER_W, CH)],
                            idx_all.at[pl.ds(0, CH)])
            fire_gather(0, 0)
            wait_gather(0)
            pltpu.sync_copy(rows_v.at[0], opt.at[pl.ds(wid * POS_PER_W, CH)])

    return k


def _feat_gather(ih, ip, ff):
    per_w = HIST // NW
    nch = per_w // CH

    @functools.partial(
        pl.kernel,
        mesh=_mesh(),
        out_type=(
            jax.ShapeDtypeStruct((FD, L, B), jnp.float32),
            jax.ShapeDtypeStruct((FD, B), jnp.float32),
        ),
        scratch_types=[
            pltpu.VMEM((per_w + CH,), jnp.int32),
            pltpu.VMEM((FD, per_w + CH), jnp.int32),
            pltpu.VMEM((FD, per_w + CH), jnp.float32),
            pltpu.SemaphoreType.DMA,
            pltpu.SemaphoreType.DMA,
        ],
    )
    def k(ih_hbm, ip_hbm, ft_hbm, ohf, opf, idx_all, idxf_v, featc_v,
          gsem, wsem):
        wid = lax.axis_index("s") * NC + lax.axis_index("c")
        hbase = wid * per_w

        def lb(c):
            base = hbase + c * CH
            l_c = base // B
            return l_c, base - l_c * B

        def idx_fetch(c, carry):
            l_c, b0 = lb(c)
            pltpu.async_copy(ih_hbm.at[l_c, 0, pl.ds(b0, CH)],
                             idx_all.at[pl.ds(c * CH, CH)], gsem)
            return carry

        lax.fori_loop(0, nch, idx_fetch, 0)
        pltpu.async_copy(ip_hbm.at[pl.ds(wid * POS_PER_W, CH)],
                         idx_all.at[pl.ds(nch * CH, CH)], gsem)

        def idx_drain(c, carry):
            pltpu.make_async_copy(ip_hbm.at[pl.ds(0, CH)],
                                  idx_all.at[pl.ds(0, CH)], gsem).wait()
            return carry

        lax.fori_loop(0, nch + 1, idx_drain, 0)

        def comp_fire(c, carry):
            for q in range(CH // 16):
                s4 = idx_all[pl.ds(c * CH + q * 16, 16)] * FD
                for j in range(FD):
                    idxf_v[j, pl.ds(c * CH + q * 16, 16)] = s4 + j
            for j in range(FD):
                pltpu.async_copy(
                    ft_hbm.at[idxf_v.at[j, pl.ds(c * CH, CH)]],
                    featc_v.at[j, pl.ds(c * CH, CH)], gsem)
            return carry

        lax.fori_loop(0, nch + 1, comp_fire, 0)

        def g_drain(c, carry):
            for j in range(FD):
                pltpu.make_async_copy(ft_hbm.at[pl.ds(0, CH)],
                                      featc_v.at[j, pl.ds(0, CH)],
                                      gsem).wait()
            return carry

        lax.fori_loop(0, nch + 1, g_drain, 0)

        def wb(c, carry):
            l_c, b0 = lb(c)
            for j in range(FD):
                pltpu.async_copy(featc_v.at[j, pl.ds(c * CH, CH)],
                                 ohf.at[j, l_c, pl.ds(b0, CH)], wsem)
            return carry

        lax.fori_loop(0, nch, wb, 0)
        for j in range(FD):
            pltpu.async_copy(featc_v.at[j, pl.ds(nch * CH, CH)],
                             opf.at[j, pl.ds(wid * POS_PER_W, CH)], wsem)

        def wb_drain(c, carry):
            for j in range(FD):
                pltpu.make_async_copy(featc_v.at[j, pl.ds(0, CH)],
                                      opf.at[j, pl.ds(0, CH)], wsem).wait()
            return carry

        lax.fori_loop(0, nch + 1, wb_drain, 0)

    return k(ih, ip, ff)


def _dot_bf16(a, w):
    return jax.lax.dot(a.astype(jnp.bfloat16), w.astype(jnp.bfloat16),
                       preferred_element_type=jnp.float32)


def _item_tower_block(x, c, W1t, b1, W2, b2, W3, b3):
    h = _dot_bf16(x, W1t) + c + b1
    h = jnp.maximum(h, 0.0)
    h = _dot_bf16(h, W2) + b2
    h = jnp.maximum(h, 0.0)
    e = _dot_bf16(h, W3) + b3
    n = jnp.sqrt(jnp.sum(e * e, axis=-1, keepdims=True))
    return e / jnp.maximum(n, 1e-12)


def _tower_pool_body(g_ref, f_ref, r_ref, m_ref, W1t_ref, W1f_ref, b1_ref,
                     W2_ref, b2_ref, W3_ref, b3_ref, out_ref, *, lo, nsl):
    bb = g_ref.shape[1]
    x = g_ref[...].reshape(nsl * bb, TD)
    W1f = W1f_ref[...]
    c3 = f_ref[0][lo:lo + nsl, :, None] * W1f[0][None, None, :]
    for j in range(1, FD):
        c3 = c3 + f_ref[j][lo:lo + nsl, :, None] * W1f[j][None, None, :]
    e = _item_tower_block(x, c3.reshape(nsl * bb, 256), W1t_ref[...],
                          b1_ref[...], W2_ref[...], b2_ref[...], W3_ref[...],
                          b3_ref[...])
    e3 = e.reshape(nsl, bb, 64)
    w = r_ref[...] * m_ref[...]
    wn = w / (jnp.sum(w, axis=1, keepdims=True) + 1e-8)
    wnT = jnp.transpose(wn[:, lo:lo + nsl])
    out_ref[...] = jnp.sum(e3 * wnT[:, :, None], axis=0)


def _pos_tower_body(g_ref, f_ref, W1t_ref, W1f_ref, b1_ref, W2_ref, b2_ref,
                    W3_ref, b3_ref, out_ref):
    W1f = W1f_ref[...]
    c = f_ref[0][:, None] * W1f[0][None, :]
    for j in range(1, FD):
        c = c + f_ref[j][:, None] * W1f[j][None, :]
    out_ref[...] = _item_tower_block(
        g_ref[...], c, W1t_ref[...], b1_ref[...],
        W2_ref[...], b2_ref[...], W3_ref[...], b3_ref[...])


def _final_body(*refs):
    (pe_ref, U1_ref, ub1_ref, U2_ref, ub2_ref, out_ref) = refs[-6:]
    p = refs[0][...]
    for r in refs[1:-6]:
        p = p + r[...]
    hp = jax.lax.Precision.HIGHEST
    h = jnp.maximum(
        jax.lax.dot(p, U1_ref[...], precision=hp) + ub1_ref[...], 0.0)
    u = jax.lax.dot(h, U2_ref[...], precision=hp) + ub2_ref[...]
    n = jnp.sqrt(jnp.sum(u * u, axis=-1, keepdims=True))
    u = u / jnp.maximum(n, 1e-12)
    out_ref[...] = jax.lax.dot(u, pe_ref[...], precision=hp) * TEMP_INV


def _full(spec):
    return pl.BlockSpec(spec, lambda i: tuple(0 for _ in spec))


def _tower_pool(g3, f3, r, m, W1t, W1f, b1, W2, b2, W3, b3, *, lo, nsl):
    BB = 128
    grid = B // BB
    return pl.pallas_call(
        functools.partial(_tower_pool_body, lo=lo, nsl=nsl),
        grid=(grid,),
        in_specs=[
            pl.BlockSpec((nsl, BB, TD), lambda i: (0, i, 0)),
            pl.BlockSpec((FD, L, BB), lambda i: (0, 0, i)),
            pl.BlockSpec((BB, L), lambda i: (i, 0)),
            pl.BlockSpec((BB, L), lambda i: (i, 0)),
            _full((TD, 256)), _full((FD, 256)), _full((256,)),
            _full((256, 128)), _full((128,)),
            _full((128, 64)), _full((64,)),
        ],
        out_specs=pl.BlockSpec((BB, 64), lambda i: (i, 0)),
        out_shape=jax.ShapeDtypeStruct((B, 64), jnp.float32),
    )(g3, f3, r, m, W1t, W1f, b1, W2, b2, W3, b3)


def _pos_tower(gp, fp, W1t, W1f, b1, W2, b2, W3, b3):
    BB = 512
    return pl.pallas_call(
        _pos_tower_body,
        grid=(B // BB,),
        in_specs=[
            pl.BlockSpec((BB, TD), lambda i: (i, 0)),
            pl.BlockSpec((FD, BB), lambda i: (0, i)),
            _full((TD, 256)), _full((FD, 256)), _full((256,)),
            _full((256, 128)), _full((128,)),
            _full((128, 64)), _full((64,)),
        ],
        out_specs=pl.BlockSpec((BB, 64), lambda i: (i, 0)),
        out_shape=jax.ShapeDtypeStruct((B, 64), jnp.float32),
    )(gp, fp, W1t, W1f, b1, W2, b2, W3, b3)


def _final(pooled_parts, pos_emb_t, U1, ub1, U2, ub2):
    BB = 512
    return pl.pallas_call(
        _final_body,
        grid=(B // BB,),
        in_specs=[pl.BlockSpec((BB, 64), lambda i: (i, 0))
                  for _ in pooled_parts] + [
            _full((64, B)),
            _full((64, 128)), _full((128,)),
            _full((128, 64)), _full((64,)),
        ],
        out_specs=pl.BlockSpec((BB, B), lambda i: (i, 0)),
        out_shape=jax.ShapeDtypeStruct((B, B), jnp.float32),
    )(*pooled_parts, pos_emb_t, U1, ub1, U2, ub2)


SPLITS = (25, 25)


def kernel(history_items, history_mask, history_ratings, pos_item, title_table,
           feat_table, W1, b1, W2, b2, W3, b3, U1, ub1, U2, ub2):
    ih = history_items.astype(jnp.int32)
    idx_pos = pos_item.astype(jnp.int32)
    W1t = W1[:TD]
    W1f = W1[TD:]
    ff = feat_table.reshape(-1)
    ihT = _idx_transpose(ih)

    hts = []
    pt = None
    lo = 0
    for si, nsl in enumerate(SPLITS):
        if si == 0:
            ht, pt = _make_title_gather(nsl, lo, True)(ihT, idx_pos,
                                                       title_table)
            hf, pf = _feat_gather(ihT, idx_pos, ff)
        else:
            ht = _make_title_gather(nsl, lo, False)(ihT, title_table)[0]
        hts.append((ht, lo, nsl))
        lo += nsl

    parts = [_tower_pool(ht, hf, history_ratings, history_mask, W1t, W1f, b1,
                         W2, b2, W3, b3, lo=lo, nsl=nsl)
             for ht, lo, nsl in hts]

    pos_emb = _pos_tower(pt, pf, W1t, W1f, b1, W2, b2, W3, b3)
    return _final(parts, pos_emb.T, U1, ub1, U2, ub2)


# speedup vs baseline: 1.0955x; 1.0955x over previous
import functools

import jax
import jax.numpy as jnp
from jax import lax
from jax.experimental import pallas as pl
from jax.experimental.pallas import tpu as pltpu
from jax.experimental.pallas import tpu_sc as plsc

TEMP_INV = 1.0 / 0.07
B, L, V, TD, FD = 4096, 50, 100000, 384, 4
FDP = 16
HIST = B * L
NC, NS = 2, 16
NW = NC * NS
CH = 128
HIST_PER_W = HIST // NW
POS_PER_W = B // NW
N_HCHUNK = HIST_PER_W // CH


def _mesh():
    return plsc.VectorSubcoreMesh(core_axis_name="c", subcore_axis_name="s")


def _idx_transpose(ih):
    BB = 512

    def body(i_ref, o_ref):
        o_ref[...] = jnp.transpose(i_ref[...])[:, None, :]

    return pl.pallas_call(
        body,
        grid=(B // BB,),
        in_specs=[pl.BlockSpec((BB, L), lambda i: (i, 0))],
        out_specs=pl.BlockSpec((L, 1, BB), lambda i: (0, 0, i)),
        out_shape=jax.ShapeDtypeStruct((L, 1, B), jnp.int32),
    )(ih)


def _feat_flatten(ftT):
    def body(i_ref, o_ref):
        for j in range(FD):
            o_ref[pl.ds(j * V, V)] = i_ref[j, :]

    return pl.pallas_call(
        body,
        grid=(1,),
        in_specs=[pl.BlockSpec((FD, V), lambda i: (0, 0))],
        out_specs=pl.BlockSpec((FD * V,), lambda i: (0,)),
        out_shape=jax.ShapeDtypeStruct((FD * V,), jnp.float32),
    )(ftT)


def _make_title_gather(nslots, lo, with_pos):
    per_w = nslots * B // NW
    nch = per_w // CH
    out_type = [jax.ShapeDtypeStruct((nslots, B, TD), jnp.float32)]
    if with_pos:
        out_type += [jax.ShapeDtypeStruct((B, TD), jnp.float32)]

    @functools.partial(
        pl.kernel,
        mesh=_mesh(),
        out_type=tuple(out_type),
        scratch_types=[
            pltpu.VMEM((per_w,), jnp.int32),
            pltpu.VMEM((2, CH, TD), jnp.float32),
            pltpu.SemaphoreType.DMA,
            pltpu.SemaphoreType.DMA,
            pltpu.SemaphoreType.DMA,
            pltpu.SemaphoreType.DMA,
        ],
    )
    def k(ih_hbm, *rest):
        if with_pos:
            (ip_hbm, tt_hbm, oht, opt, idx_all, rows_v,
             gsem0, gsem1, wsem0, wsem1) = rest
        else:
            (tt_hbm, oht, idx_all, rows_v, gsem0, gsem1, wsem0, wsem1) = rest
        wid = lax.axis_index("s") * NC + lax.axis_index("c")
        hbase = wid * per_w
        gsem = (gsem0, gsem1)
        wsem = (wsem0, wsem1)

        def lb(c):
            base = hbase + c * CH
            l_c = base // B
            return l_c, base - l_c * B

        for c in range(nch):
            l_c, b0 = lb(c)
            pltpu.async_copy(ih_hbm.at[lo + l_c, 0, pl.ds(b0, CH)],
                             idx_all.at[pl.ds(c * CH, CH)], gsem0)
        for c in range(nch):
            pltpu.make_async_copy(ih_hbm.at[0, 0, pl.ds(0, CH)],
                                  idx_all.at[pl.ds(0, CH)], gsem0).wait()

        def fire_gather(c, b):
            pltpu.async_copy(tt_hbm.at[idx_all.at[pl.ds(c * CH, CH)]],
                             rows_v.at[b], gsem[b])

        def wait_gather(b):
            pltpu.make_async_copy(tt_hbm.at[pl.ds(0, CH)], rows_v.at[b],
                                  gsem[b]).wait()

        def fire_wb(c, b):
            l_c, b0 = lb(c)
            pltpu.async_copy(rows_v.at[b], oht.at[l_c, pl.ds(b0, CH)],
                             wsem[b])

        def wait_wb(b):
            pltpu.make_async_copy(rows_v.at[b], oht.at[0, pl.ds(0, CH)],
                                  wsem[b]).wait()

        fire_gather(0, 0)

        def body(g2, carry):
            for b in (0, 1):
                cg = 2 * g2 + b
                nb = 1 - b

                @pl.when(cg + 1 < nch)
                def _fire_next():
                    @pl.when(cg >= 1)
                    def _drain_prev_wb():
                        wait_wb(nb)

                    fire_gather(cg + 1, nb)

                wait_gather(b)
                fire_wb(cg, b)
            return carry

        lax.fori_loop(0, nch // 2, body, 0)
        if nch % 2:
            wait_gather(0)
            fire_wb(nch - 1, 0)
        wait_wb(0)
        if nch >= 2:
            wait_wb(1)

        if with_pos:
            pltpu.sync_copy(ip_hbm.at[pl.ds(wid * POS_PER_W, CH)],
                            idx_all.at[pl.ds(0, CH)])
            fire_gather(0, 0)
            wait_gather(0)
            pltpu.sync_copy(rows_v.at[0], opt.at[pl.ds(wid * POS_PER_W, CH)])

    return k


def _feat_gather(ih, ip, ff):
    per_w = HIST // NW
    nch = per_w // CH

    @functools.partial(
        pl.kernel,
        mesh=_mesh(),
        out_type=(
            jax.ShapeDtypeStruct((FD, L, B), jnp.float32),
            jax.ShapeDtypeStruct((FD, B), jnp.float32),
        ),
        scratch_types=[
            pltpu.VMEM((per_w + CH,), jnp.int32),
            pltpu.VMEM((FD, per_w + CH), jnp.int32),
            pltpu.VMEM((FD, per_w + CH), jnp.float32),
            pltpu.SemaphoreType.DMA,
            pltpu.SemaphoreType.DMA,
        ],
    )
    def k(ih_hbm, ip_hbm, ft_hbm, ohf, opf, idx_all, idxf_v, featc_v,
          gsem, wsem):
        wid = lax.axis_index("s") * NC + lax.axis_index("c")
        hbase = wid * per_w

        def lb(c):
            base = hbase + c * CH
            l_c = base // B
            return l_c, base - l_c * B

        def idx_fetch(c, carry):
            l_c, b0 = lb(c)
            pltpu.async_copy(ih_hbm.at[l_c, 0, pl.ds(b0, CH)],
                             idx_all.at[pl.ds(c * CH, CH)], gsem)
            return carry

        lax.fori_loop(0, nch, idx_fetch, 0)
        pltpu.async_copy(ip_hbm.at[pl.ds(wid * POS_PER_W, CH)],
                         idx_all.at[pl.ds(nch * CH, CH)], gsem)

        def idx_drain(c, carry):
            pltpu.make_async_copy(ip_hbm.at[pl.ds(0, CH)],
                                  idx_all.at[pl.ds(0, CH)], gsem).wait()
            return carry

        lax.fori_loop(0, nch + 1, idx_drain, 0)

        def comp_fire(c, carry):
            for q in range(CH // 16):
                s = idx_all[pl.ds(c * CH + q * 16, 16)]
                for j in range(FD):
                    idxf_v[j, pl.ds(c * CH + q * 16, 16)] = s + j * V
            for j in range(FD):
                pltpu.async_copy(
                    ft_hbm.at[idxf_v.at[j, pl.ds(c * CH, CH)]],
                    featc_v.at[j, pl.ds(c * CH, CH)], gsem)
            return carry

        lax.fori_loop(0, nch + 1, comp_fire, 0)

        def g_drain(c, carry):
            for j in range(FD):
                pltpu.make_async_copy(ft_hbm.at[pl.ds(0, CH)],
                                      featc_v.at[j, pl.ds(0, CH)],
                                      gsem).wait()
            return carry

        lax.fori_loop(0, nch + 1, g_drain, 0)

        def wb(c, carry):
            l_c, b0 = lb(c)
            for j in range(FD):
                pltpu.async_copy(featc_v.at[j, pl.ds(c * CH, CH)],
                                 ohf.at[j, l_c, pl.ds(b0, CH)], wsem)
            return carry

        lax.fori_loop(0, nch, wb, 0)
        for j in range(FD):
            pltpu.async_copy(featc_v.at[j, pl.ds(nch * CH, CH)],
                             opf.at[j, pl.ds(wid * POS_PER_W, CH)], wsem)

        def wb_drain(c, carry):
            for j in range(FD):
                pltpu.make_async_copy(featc_v.at[j, pl.ds(0, CH)],
                                      opf.at[j, pl.ds(0, CH)], wsem).wait()
            return carry

        lax.fori_loop(0, nch + 1, wb_drain, 0)

    return k(ih, ip, ff)


def _dot_bf16(a, w):
    return jax.lax.dot(a.astype(jnp.bfloat16), w.astype(jnp.bfloat16),
                       preferred_element_type=jnp.float32)


def _item_tower_block(x, c, W1t, b1, W2, b2, W3, b3):
    h = _dot_bf16(x, W1t) + c + b1
    h = jnp.maximum(h, 0.0)
    h = _dot_bf16(h, W2) + b2
    h = jnp.maximum(h, 0.0)
    e = _dot_bf16(h, W3) + b3
    n = jnp.sqrt(jnp.sum(e * e, axis=-1, keepdims=True))
    return e / jnp.maximum(n, 1e-12)


def _tower_pool_body(g_ref, f_ref, r_ref, m_ref, W1t_ref, W1f_ref, b1_ref,
                     W2_ref, b2_ref, W3_ref, b3_ref, out_ref, *, lo, nsl):
    bb = g_ref.shape[1]
    x = g_ref[...].reshape(nsl * bb, TD)
    W1f = W1f_ref[...]
    c3 = f_ref[0][lo:lo + nsl, :, None] * W1f[0][None, None, :]
    for j in range(1, FD):
        c3 = c3 + f_ref[j][lo:lo + nsl, :, None] * W1f[j][None, None, :]
    e = _item_tower_block(x, c3.reshape(nsl * bb, 256), W1t_ref[...],
                          b1_ref[...], W2_ref[...], b2_ref[...], W3_ref[...],
                          b3_ref[...])
    e3 = e.reshape(nsl, bb, 64)
    w = r_ref[...] * m_ref[...]
    wn = w / (jnp.sum(w, axis=1, keepdims=True) + 1e-8)
    wnT = jnp.transpose(wn[:, lo:lo + nsl])
    out_ref[...] = jnp.sum(e3 * wnT[:, :, None], axis=0)


def _pos_tower_body(g_ref, f_ref, W1t_ref, W1f_ref, b1_ref, W2_ref, b2_ref,
                    W3_ref, b3_ref, out_ref):
    W1f = W1f_ref[...]
    c = f_ref[0][:, None] * W1f[0][None, :]
    for j in range(1, FD):
        c = c + f_ref[j][:, None] * W1f[j][None, :]
    out_ref[...] = _item_tower_block(
        g_ref[...], c, W1t_ref[...], b1_ref[...],
        W2_ref[...], b2_ref[...], W3_ref[...], b3_ref[...])


def _final_body(*refs):
    (pe_ref, U1_ref, ub1_ref, U2_ref, ub2_ref, out_ref) = refs[-6:]
    p = refs[0][...]
    for r in refs[1:-6]:
        p = p + r[...]
    hp = jax.lax.Precision.HIGHEST
    h = jnp.maximum(
        jax.lax.dot(p, U1_ref[...], precision=hp) + ub1_ref[...], 0.0)
    u = jax.lax.dot(h, U2_ref[...], precision=hp) + ub2_ref[...]
    n = jnp.sqrt(jnp.sum(u * u, axis=-1, keepdims=True))
    u = u / jnp.maximum(n, 1e-12)
    out_ref[...] = jax.lax.dot(u, pe_ref[...], precision=hp) * TEMP_INV


def _full(spec):
    return pl.BlockSpec(spec, lambda i: tuple(0 for _ in spec))


def _tower_pool(g3, f3, r, m, W1t, W1f, b1, W2, b2, W3, b3, *, lo, nsl):
    BB = 128
    grid = B // BB
    return pl.pallas_call(
        functools.partial(_tower_pool_body, lo=lo, nsl=nsl),
        grid=(grid,),
        in_specs=[
            pl.BlockSpec((nsl, BB, TD), lambda i: (0, i, 0)),
            pl.BlockSpec((FD, L, BB), lambda i: (0, 0, i)),
            pl.BlockSpec((BB, L), lambda i: (i, 0)),
            pl.BlockSpec((BB, L), lambda i: (i, 0)),
            _full((TD, 256)), _full((FD, 256)), _full((256,)),
            _full((256, 128)), _full((128,)),
            _full((128, 64)), _full((64,)),
        ],
        out_specs=pl.BlockSpec((BB, 64), lambda i: (i, 0)),
        out_shape=jax.ShapeDtypeStruct((B, 64), jnp.float32),
    )(g3, f3, r, m, W1t, W1f, b1, W2, b2, W3, b3)


def _pos_tower(gp, fp, W1t, W1f, b1, W2, b2, W3, b3):
    BB = 512
    return pl.pallas_call(
        _pos_tower_body,
        grid=(B // BB,),
        in_specs=[
            pl.BlockSpec((BB, TD), lambda i: (i, 0)),
            pl.BlockSpec((FD, BB), lambda i: (0, i)),
            _full((TD, 256)), _full((FD, 256)), _full((256,)),
            _full((256, 128)), _full((128,)),
            _full((128, 64)), _full((64,)),
        ],
        out_specs=pl.BlockSpec((BB, 64), lambda i: (i, 0)),
        out_shape=jax.ShapeDtypeStruct((B, 64), jnp.float32),
    )(gp, fp, W1t, W1f, b1, W2, b2, W3, b3)


def _final(pooled_parts, pos_emb_t, U1, ub1, U2, ub2):
    BB = 512
    return pl.pallas_call(
        _final_body,
        grid=(B // BB,),
        in_specs=[pl.BlockSpec((BB, 64), lambda i: (i, 0))
                  for _ in pooled_parts] + [
            _full((64, B)),
            _full((64, 128)), _full((128,)),
            _full((128, 64)), _full((64,)),
        ],
        out_specs=pl.BlockSpec((BB, B), lambda i: (i, 0)),
        out_shape=jax.ShapeDtypeStruct((B, B), jnp.float32),
    )(*pooled_parts, pos_emb_t, U1, ub1, U2, ub2)


SPLITS = (25, 25)


def kernel(history_items, history_mask, history_ratings, pos_item, title_table,
           feat_table, W1, b1, W2, b2, W3, b3, U1, ub1, U2, ub2):
    ih = history_items.astype(jnp.int32)
    idx_pos = pos_item.astype(jnp.int32)
    W1t = W1[:TD]
    W1f = W1[TD:]
    ff = _feat_flatten(feat_table.T)
    ihT = _idx_transpose(ih)

    hts = []
    pt = None
    lo = 0
    for si, nsl in enumerate(SPLITS):
        if si == 0:
            ht, pt = _make_title_gather(nsl, lo, True)(ihT, idx_pos,
                                                       title_table)
            tok = (ht[0, 0, 0] * 0.0).astype(jnp.int32)
            hf, pf = _feat_gather(ihT, idx_pos + tok, ff)
            ihT = ihT + (hf[0, 0, 0] * 0.0).astype(jnp.int32)
        else:
            ht = _make_title_gather(nsl, lo, False)(ihT, title_table)[0]
        hts.append((ht, lo, nsl))
        lo += nsl

    parts = [_tower_pool(ht, hf, history_ratings, history_mask, W1t, W1f, b1,
                         W2, b2, W3, b3, lo=lo, nsl=nsl)
             for ht, lo, nsl in hts]

    pos_emb = _pos_tower(pt, pf, W1t, W1f, b1, W2, b2, W3, b3)
    return _final(parts, pos_emb.T, U1, ub1, U2, ub2)
